# baseline (speedup 1.0000x reference)
import functools
import math

import jax
import jax.numpy as jnp
from jax import lax
from jax.experimental import pallas as pl
from jax.experimental.pallas import tpu as pltpu
from jax.experimental.pallas import tpu_sc as plsc

D_MODEL = 64
SCALE = math.sqrt(D_MODEL)


def _make_lookup(B):
    info = plsc.get_sparse_core_info()
    NC, NS, L = info.num_cores, info.num_subcores, info.num_lanes
    NW = NC * NS
    b_per_w = B // NW
    C = 512
    KG = C // 128
    n_chunks = b_per_w // C
    assert b_per_w % C == 0
    mesh = plsc.VectorSubcoreMesh(core_axis_name="c", subcore_axis_name="s")

    @functools.partial(
        pl.kernel,
        mesh=mesh,
        out_type=jax.ShapeDtypeStruct((B, D_MODEL), jnp.float32),
        scratch_types=[
            pltpu.VMEM((C,), jnp.int32),
            pltpu.VMEM((C, D_MODEL), jnp.float32),
            pltpu.SemaphoreType.DMA,
        ],
        compiler_params=pltpu.CompilerParams(use_tc_tiling_on_sc=False),
    )
    def lookup(x_hbm, table_hbm, out_hbm, idx_v, rows_v, sem):
        wid = lax.axis_index("s") * NC + lax.axis_index("c")
        base = wid * b_per_w

        def chunk_body(g, carry):
            off = base + g * C
            pltpu.sync_copy(x_hbm.at[pl.ds(off, C)], idx_v)
            copies = []
            for j in range(KG):
                copies.append(
                    pltpu.async_copy(
                        table_hbm.at[idx_v.at[pl.ds(j * 128, 128)]],
                        rows_v.at[pl.ds(j * 128, 128)],
                        sem,
                    )
                )
            for c in copies:
                c.wait()

            def scale_row(r, carry2):
                for cc in range(D_MODEL // L):
                    rows_v[r, pl.ds(cc * L, L)] = (
                        rows_v[r, pl.ds(cc * L, L)] * SCALE
                    )
                return carry2

            lax.fori_loop(0, C, scale_row, 0, unroll=2)
            pltpu.sync_copy(rows_v, out_hbm.at[pl.ds(off, C)])
            return carry

        lax.fori_loop(0, n_chunks, chunk_body, 0)

    return lookup


def kernel(x, table):
    shape = x.shape
    B = x.size
    flat = _make_lookup(B)(jnp.reshape(x, (B,)), table)
    return jnp.reshape(flat, shape + (D_MODEL,))


# speedup vs baseline: 1.0883x; 1.0883x over previous
import functools
import math

import jax
import jax.numpy as jnp
from jax import lax
from jax.experimental import pallas as pl
from jax.experimental.pallas import tpu as pltpu
from jax.experimental.pallas import tpu_sc as plsc

D_MODEL = 64
SCALE = math.sqrt(D_MODEL)


def _make_lookup(B):
    info = plsc.get_sparse_core_info()
    NC, NS, L = info.num_cores, info.num_subcores, info.num_lanes
    NW = NC * NS
    b_per_w = B // NW
    C = 512
    KG = C // 128
    NB = 2
    n_chunks = b_per_w // C
    assert b_per_w % C == 0 and n_chunks % NB == 0
    mesh = plsc.VectorSubcoreMesh(core_axis_name="c", subcore_axis_name="s")

    @functools.partial(
        pl.kernel,
        mesh=mesh,
        out_type=jax.ShapeDtypeStruct((B, D_MODEL), jnp.float32),
        scratch_types=[
            pltpu.VMEM((b_per_w,), jnp.int32),
            pltpu.VMEM((NB, C, D_MODEL), jnp.float32),
            pltpu.SemaphoreType.DMA((NB,)),
            pltpu.SemaphoreType.DMA((NB,)),
        ],
        compiler_params=pltpu.CompilerParams(use_tc_tiling_on_sc=False),
    )
    def lookup(x_hbm, table_hbm, out_hbm, idx_v, rows_v, gsem, wsem):
        wid = lax.axis_index("s") * NC + lax.axis_index("c")
        base = wid * b_per_w
        pltpu.sync_copy(x_hbm.at[pl.ds(base, b_per_w)], idx_v)

        def fire_gathers(t, b):
            for j in range(KG):
                pltpu.async_copy(
                    table_hbm.at[idx_v.at[pl.ds(t * C + j * 128, 128)]],
                    rows_v.at[b, pl.ds(j * 128, 128)],
                    gsem.at[b],
                )

        fire_gathers(0, 0)

        @pl.loop(0, n_chunks, step=NB)
        def outer(t0):
            for b in range(NB):
                t = t0 + b
                b1 = (b + 1) % NB

                @pl.when(t >= 1)
                def _drain_write():
                    pltpu.make_async_copy(
                        rows_v.at[b1], out_hbm.at[pl.ds(base, C)], wsem.at[b1]
                    ).wait()

                @pl.when(t + 1 < n_chunks)
                def _prefetch():
                    fire_gathers(t + 1, b1)

                pltpu.make_async_copy(
                    table_hbm.at[pl.ds(0, C)], rows_v.at[b], gsem.at[b]
                ).wait()

                @plsc.parallel_loop(0, C, 1, unroll=8)
                def _scale(r):
                    for cc in range(D_MODEL // L):
                        rows_v[b, r, pl.ds(cc * L, L)] = (
                            rows_v[b, r, pl.ds(cc * L, L)] * SCALE
                        )

                pltpu.async_copy(
                    rows_v.at[b], out_hbm.at[pl.ds(base + t * C, C)], wsem.at[b]
                )

        bl = (n_chunks - 1) % NB
        pltpu.make_async_copy(
            rows_v.at[bl], out_hbm.at[pl.ds(base, C)], wsem.at[bl]
        ).wait()

    return lookup


def kernel(x, table):
    shape = x.shape
    B = x.size
    flat = _make_lookup(B)(jnp.reshape(x, (B,)), table)
    return jnp.reshape(flat, shape + (D_MODEL,))
